# Initial kernel scaffold; baseline (speedup 1.0000x reference)
import jax
import jax.numpy as jnp
from jax.experimental import pallas as pl


def kernel(features, img_feat, edge_index, params):
    raise NotImplementedError("write your pallas kernel here")


# speedup vs baseline: 1.0873x; 1.0873x over previous
import functools
import math

import jax
import jax.numpy as jnp
from jax.experimental import pallas as pl


def _proj_kernel(x_ref, w_ref, b_ref, o_ref):
    o_ref[...] = (
        jnp.dot(x_ref[...], w_ref[...], preferred_element_type=jnp.float32)
        + b_ref[...]
    )


def _project(x, w, b):
    n, din = x.shape
    dout = w.shape[0]
    return pl.pallas_call(
        _proj_kernel,
        out_shape=jax.ShapeDtypeStruct((n, dout), jnp.float32),
    )(x, w.T, b[None, :])


def _transformer_conv(x, edge_index, p):
    n = x.shape[0]
    src, dst = edge_index[0], edge_index[1]
    q = _project(x, p['Wq'], p['bq'])
    k = _project(x, p['Wk'], p['bk'])
    v = _project(x, p['Wv'], p['bv'])
    d = q.shape[-1]
    score = jnp.sum(q[dst] * k[src], axis=-1) / math.sqrt(d)
    m = jax.ops.segment_max(score, dst, num_segments=n)
    m = jnp.where(jnp.isfinite(m), m, 0.0)
    e = jnp.exp(score - m[dst])
    ssum = jax.ops.segment_sum(e, dst, num_segments=n)
    alpha = e / (ssum[dst] + 1e-16)
    out = jax.ops.segment_sum(v[src] * alpha[:, None], dst, num_segments=n)
    skip = _project(x, p['Wskip'], p['bskip'])
    beta = jax.nn.sigmoid(
        jnp.concatenate([out, skip, out - skip], axis=-1) @ p['Wbeta'].T)
    return beta * skip + (1.0 - beta) * out


def kernel(features, img_feat, edge_index, params):
    tc = lambda x, name: _transformer_conv(x, edge_index, params[name])
    act = jax.nn.elu
    h1 = act(tc(features, 'conv1'))
    h2 = tc(h1, 'conv2')
    h3 = act(tc(h2, 'conv3'))
    h4 = tc(h3, 'conv4')
    img1 = act(tc(img_feat, 'imgconv1'))
    img2 = tc(img1, 'imgconv2')
    img3 = act(tc(img2, 'imgconv3'))
    img4 = tc(img3, 'imgconv4')
    concat = jnp.concatenate([h2, img2], axis=1)
    combine = act(tc(concat, 'neck'))
    c2 = tc(combine, 'neck2')
    c3 = act(tc(c2, 'c3'))
    c4 = tc(c3, 'c4')
    return (h2, img2, c2, h4, img4, c4)


# speedup vs baseline: 7.3834x; 6.7903x over previous
import functools
import math

import jax
import jax.numpy as jnp
from jax import lax
from jax.experimental import pallas as pl
from jax.experimental.pallas import tpu as pltpu
from jax.experimental.pallas import tpu_sc as plsc

N = 10000
E = 320000
NUM_TILES = 32
E_PER_TILE = E // NUM_TILES
CHUNK = 400
N_CHUNKS = E_PER_TILE // CHUNK
ZROWS = 1000


def _proj_body(x_ref, w_ref, b_ref, q_ref, k_ref, s_ref, mp_ref, *v_refs,
               d):
    y = jnp.dot(x_ref[...], w_ref[...],
                preferred_element_type=jnp.float32) + b_ref[...]
    q = y[:, :d]
    k = y[:, d:2 * d]
    q_ref[...] = q
    k_ref[...] = k
    s_ref[...] = y[:, 3 * d:]
    for i, v_ref in enumerate(v_refs):
        v_ref[...] = y[:, 2 * d + 32 * i:2 * d + 32 * (i + 1)]
    qn = jnp.sqrt(jnp.sum(q * q, axis=1, keepdims=True))
    kmax = jnp.sqrt(jnp.max(jnp.sum(k * k, axis=1)))
    mp_ref[...] = qn * (kmax / math.sqrt(d))


def _project(x, p, d):
    wcat = jnp.concatenate(
        [p['Wq'].T, p['Wk'].T, p['Wv'].T, p['Wskip'].T], axis=1)
    bcat = jnp.concatenate(
        [p['bq'], p['bk'], p['bv'], p['bskip']])[None, :]
    n = x.shape[0]
    nparts = d // 32
    q, k, s, mp, *vparts = pl.pallas_call(
        functools.partial(_proj_body, d=d),
        out_shape=(
            jax.ShapeDtypeStruct((n, d), jnp.float32),
            jax.ShapeDtypeStruct((n, d), jnp.float32),
            jax.ShapeDtypeStruct((n, d), jnp.float32),
            jax.ShapeDtypeStruct((n, 1), jnp.float32),
        ) + tuple(jax.ShapeDtypeStruct((n, 32), jnp.float32)
                  for _ in range(nparts)),
    )(x, wcat, bcat)
    return q, k, vparts, s, mp.reshape(-1)


def _combine_body(o_ref, sk_ref, wa_ref, wb_ref, y_ref, *, act):
    out = o_ref[0] + o_ref[1]
    sk = sk_ref[...]
    logit = jnp.sum(out * wa_ref[...] + sk * wb_ref[...], axis=1,
                    keepdims=True)
    beta = jax.nn.sigmoid(logit)
    y = beta * sk + (1.0 - beta) * out
    if act:
        y = jnp.where(y > 0, y, jnp.exp(jnp.minimum(y, 0.0)) - 1.0)
    y_ref[...] = y


def _combine(out2, skip, wbeta, d, act):
    wa = (wbeta[0, :d] + wbeta[0, 2 * d:3 * d])[None, :]
    wb = (wbeta[0, d:2 * d] - wbeta[0, 2 * d:3 * d])[None, :]
    n = skip.shape[0]
    return pl.pallas_call(
        functools.partial(_combine_body, act=act),
        out_shape=jax.ShapeDtypeStruct((n, d), jnp.float32),
    )(out2, skip, wa, wb)


_MESH = plsc.VectorSubcoreMesh(core_axis_name="c", subcore_axis_name="s")


@functools.lru_cache(maxsize=None)
def _make_edge_score_kernel(d):
    @functools.partial(
        pl.kernel, mesh=_MESH,
        compiler_params=pltpu.CompilerParams(
            needs_layout_passes=False, use_tc_tiling_on_sc=False),
        out_type=(
            jax.ShapeDtypeStruct((E,), jnp.float32),
            jax.ShapeDtypeStruct((N,), jnp.float32),
            jax.ShapeDtypeStruct((N,), jnp.float32),
        ),
        scratch_types=[
            pltpu.VMEM((N,), jnp.float32),
            pltpu.VMEM((CHUNK,), jnp.int32),
            pltpu.VMEM((5, 80), jnp.int32),
            pltpu.VMEM((5, 80), jnp.int32),
            pltpu.VMEM((CHUNK, d), jnp.float32),
            pltpu.VMEM((CHUNK, d), jnp.float32),
            pltpu.VMEM((CHUNK,), jnp.float32),
            pltpu.VMEM((256,), jnp.float32),
            pltpu.VMEM((ZROWS,), jnp.float32),
            pltpu.VMEM_SHARED((N,), jnp.float32),
            pltpu.SemaphoreType.DMA,
            pltpu.SemaphoreType.DMA,
        ],
    )
    def kern(q_hbm, k_hbm, mp_hbm, src_hbm, dst_hbm, z_hbm,
             t_hbm, ssum0_hbm, ssum1_hbm,
             mp_v, dst_v, dstw_v, srcw_v, qd_v, ks_v, t_v, tr_v, zst_v,
             ssum_sh, sem1, sem2):
        cid = lax.axis_index("c")
        sid = lax.axis_index("s")
        wid = sid * 2 + cid

        @pl.when(sid < 10)
        def _():
            pltpu.sync_copy(z_hbm, zst_v)
            pltpu.sync_copy(zst_v, ssum_sh.at[pl.ds(sid * ZROWS, ZROWS)])

        pltpu.sync_copy(mp_hbm, mp_v)
        plsc.subcore_barrier()

        ev = jnp.arange(16, dtype=jnp.int32)
        inv_sqrt_d = 1.0 / math.sqrt(d)

        def chunk_body(ch, carry):
            base = wid * E_PER_TILE + ch * CHUNK
            pltpu.sync_copy(dst_hbm.at[pl.ds(base, CHUNK)], dst_v)
            for j in range(5):
                pltpu.sync_copy(dst_hbm.at[pl.ds(base + j * 80, 80)],
                                dstw_v.at[j])
                pltpu.sync_copy(src_hbm.at[pl.ds(base + j * 80, 80)],
                                srcw_v.at[j])
            cps = []
            for j in range(5):
                cps.append(pltpu.async_copy(
                    q_hbm.at[dstw_v.at[j]],
                    qd_v.at[pl.ds(j * 80, 80)], sem1))
                cps.append(pltpu.async_copy(
                    k_hbm.at[srcw_v.at[j]],
                    ks_v.at[pl.ds(j * 80, 80)], sem2))
            for cp in cps:
                cp.wait()

            def group_body(g, c2):
                eb = g * 16
                for r in range(16):
                    row = eb + r
                    acc = (qd_v[row, pl.ds(0, 16)] *
                           ks_v[row, pl.ds(0, 16)])
                    for c4 in range(1, d // 16):
                        sl = pl.ds(c4 * 16, 16)
                        acc = acc + qd_v[row, sl] * ks_v[row, sl]
                    tr_v[pl.ds(r * 16, 16)] = acc
                acc16 = plsc.load_gather(tr_v, [ev * 16])
                for j in range(1, 16):
                    acc16 = acc16 + plsc.load_gather(tr_v, [ev * 16 + j])
                dst16 = dst_v[pl.ds(eb, 16)]
                mp16 = plsc.load_gather(mp_v, [dst16])
                t_v[pl.ds(eb, 16)] = jnp.exp(acc16 * inv_sqrt_d - mp16)
                return c2

            lax.fori_loop(0, CHUNK // 16, group_body, 0)
            pltpu.sync_copy(t_v, t_hbm.at[pl.ds(base, CHUNK)])
            for j in range(5):
                pltpu.sync_copy(t_v.at[pl.ds(j * 80, 80)],
                                ssum_sh.at[dstw_v.at[j]], add=True)
            return carry

        lax.fori_loop(0, N_CHUNKS, chunk_body, 0)
        plsc.subcore_barrier()

        @pl.when(sid < 10)
        def _():
            pltpu.sync_copy(ssum_sh.at[pl.ds(sid * ZROWS, ZROWS)], zst_v)

            @pl.when(cid == 0)
            def _():
                pltpu.sync_copy(zst_v, ssum0_hbm.at[pl.ds(sid * ZROWS, ZROWS)])

            @pl.when(cid == 1)
            def _():
                pltpu.sync_copy(zst_v, ssum1_hbm.at[pl.ds(sid * ZROWS, ZROWS)])

    return kern


@functools.lru_cache(maxsize=None)
def _make_edge_agg_kernel(d):
    nparts = d // 32

    @functools.partial(
        pl.kernel, mesh=_MESH,
        compiler_params=pltpu.CompilerParams(
            needs_layout_passes=False, use_tc_tiling_on_sc=False),
        out_type=tuple(jax.ShapeDtypeStruct((2, N, 32), jnp.float32)
                       for _ in range(nparts)),
        scratch_types=[
            pltpu.VMEM((N,), jnp.float32),
            pltpu.VMEM((N,), jnp.float32),
            pltpu.VMEM((CHUNK,), jnp.int32),
            pltpu.VMEM((5, 80), jnp.int32),
            pltpu.VMEM((5, 80), jnp.int32),
            pltpu.VMEM((CHUNK,), jnp.float32),
            pltpu.VMEM((CHUNK, 32), jnp.float32),
            pltpu.VMEM((32,), jnp.float32),
            pltpu.VMEM((250, 32), jnp.float32),
            pltpu.VMEM_SHARED((N, 32), jnp.float32),
            pltpu.SemaphoreType.DMA,
        ],
    )
    def kern(*args):
        v_hbms = args[:nparts]
        (ssum0_hbm, ssum1_hbm, src_hbm, dst_hbm, t_hbm, z_hbm) = \
            args[nparts:nparts + 6]
        out_hbms = args[nparts + 6:2 * nparts + 6]
        (ssum_v, tmp_v, dst_v, dstw_v, srcw_v, t_v, vs_v, al_v, zst_v,
         out_sh, sem) = args[2 * nparts + 6:]
        cid = lax.axis_index("c")
        sid = lax.axis_index("s")
        wid = sid * 2 + cid

        pltpu.sync_copy(ssum0_hbm, ssum_v)
        pltpu.sync_copy(ssum1_hbm, tmp_v)

        def add_body(i, carry):
            sl = pl.ds(i * 16, 16)
            ssum_v[sl] = ssum_v[sl] + tmp_v[sl]
            return carry

        lax.fori_loop(0, N // 16, add_body, 0)

        ev = jnp.arange(16, dtype=jnp.int32)

        for part in range(nparts):
            @pl.when(sid < 10)
            def _():
                pltpu.sync_copy(z_hbm, zst_v)
                for j in range(4):
                    pltpu.sync_copy(
                        zst_v, out_sh.at[pl.ds(sid * ZROWS + j * 250, 250)])

            plsc.subcore_barrier()

            def chunk_body(ch, carry, part=part):
                base = wid * E_PER_TILE + ch * CHUNK
                pltpu.sync_copy(dst_hbm.at[pl.ds(base, CHUNK)], dst_v)
                pltpu.sync_copy(t_hbm.at[pl.ds(base, CHUNK)], t_v)
                for j in range(5):
                    pltpu.sync_copy(dst_hbm.at[pl.ds(base + j * 80, 80)],
                                    dstw_v.at[j])
                    pltpu.sync_copy(src_hbm.at[pl.ds(base + j * 80, 80)],
                                    srcw_v.at[j])
                cps = [pltpu.async_copy(v_hbms[part].at[srcw_v.at[j]],
                                        vs_v.at[pl.ds(j * 80, 80)], sem)
                       for j in range(5)]
                for cp in cps:
                    cp.wait()

                def group_body(g, c2):
                    eb = g * 16
                    dst16 = dst_v[pl.ds(eb, 16)]
                    s16 = plsc.load_gather(ssum_v, [dst16])
                    al_v[pl.ds(16, 16)] = t_v[pl.ds(eb, 16)] / (s16 + 1e-16)
                    for r in range(16):
                        row = eb + r
                        splat = plsc.load_gather(
                            al_v, [jnp.full((16,), 16 + r, jnp.int32)])
                        for c4 in range(2):
                            sl = pl.ds(c4 * 16, 16)
                            vs_v[row, sl] = vs_v[row, sl] * splat
                    return c2

                lax.fori_loop(0, CHUNK // 16, group_body, 0)
                for j in range(5):
                    pltpu.sync_copy(vs_v.at[pl.ds(j * 80, 80)],
                                    out_sh.at[dstw_v.at[j]], add=True)
                return carry

            lax.fori_loop(0, N_CHUNKS, chunk_body, 0)
            plsc.subcore_barrier()

            @pl.when(sid < 10)
            def _(part=part):
                for j in range(4):
                    pltpu.sync_copy(
                        out_sh.at[pl.ds(sid * ZROWS + j * 250, 250)], zst_v)
                    pltpu.sync_copy(
                        zst_v,
                        out_hbms[part].at[cid,
                                          pl.ds(sid * ZROWS + j * 250, 250)])

    return kern


def _transformer_conv(x, src, dst, p, act):
    d = p['Wq'].shape[0]
    q, k, vparts, skip, mp = _project(x, p, d)
    z1 = jnp.zeros((ZROWS,), jnp.float32)
    z2 = jnp.zeros((250, 32), jnp.float32)
    t, ssum0, ssum1 = _make_edge_score_kernel(d)(q, k, mp, src, dst, z1)
    outs = _make_edge_agg_kernel(d)(*vparts, ssum0, ssum1, src, dst, t, z2)
    if not isinstance(outs, (tuple, list)):
        outs = (outs,)
    out2 = jnp.concatenate(outs, axis=2)
    return _combine(out2, skip, p['Wbeta'], d, act)


def kernel(features, img_feat, edge_index, params):
    src, dst = edge_index[0], edge_index[1]
    tc = lambda x, name, act: _transformer_conv(
        x, src, dst, params[name], act)
    h1 = tc(features, 'conv1', True)
    img1 = tc(img_feat, 'imgconv1', True)
    h2 = tc(h1, 'conv2', False)
    img2 = tc(img1, 'imgconv2', False)
    h3 = tc(h2, 'conv3', True)
    img3 = tc(img2, 'imgconv3', True)
    h4 = tc(h3, 'conv4', False)
    img4 = tc(img3, 'imgconv4', False)
    concat = jnp.concatenate([h2, img2], axis=1)
    combine = tc(concat, 'neck', True)
    c2 = tc(combine, 'neck2', False)
    c3 = tc(c2, 'c3', True)
    c4 = tc(c3, 'c4', False)
    return (h2, img2, c2, h4, img4, c4)
